# speedup vs baseline: 1.5040x; 1.5040x over previous
import functools
import math

import jax
import jax.numpy as jnp
from jax import lax
from jax.experimental import pallas as pl
from jax.experimental.pallas import tpu as pltpu
from jax.experimental.pallas import tpu_sc as plsc

D_MODEL = 512
SCALE = math.sqrt(512.0)

NC = 2
NS = 16
LANES = 16
NW = NC * NS

B_TOTAL = 4 * 8192
B_PER_W = B_TOTAL // NW
CHUNK = 64
N_CHUNKS = B_PER_W // CHUNK
VECS_PER_ROW = D_MODEL // LANES


def _body(table_hbm, idx_hbm, out_hbm, idx_v, rows_v, in_sems, out_sems):
    wid = lax.axis_index("s") * NC + lax.axis_index("c")
    base = wid * B_PER_W

    pltpu.sync_copy(idx_hbm.at[pl.ds(base, B_PER_W)], idx_v)

    def start_gather(c, buf):
        pltpu.async_copy(
            table_hbm.at[idx_v.at[pl.ds(c * CHUNK, CHUNK)]],
            rows_v.at[buf],
            in_sems.at[buf],
        )

    def wait_gather(c, buf):
        pltpu.make_async_copy(
            table_hbm.at[idx_v.at[pl.ds(c * CHUNK, CHUNK)]],
            rows_v.at[buf],
            in_sems.at[buf],
        ).wait()

    def wait_scatter(c, buf):
        pltpu.make_async_copy(
            rows_v.at[buf],
            out_hbm.at[pl.ds(base + c * CHUNK, CHUNK)],
            out_sems.at[buf],
        ).wait()

    def scale_buf(buf):
        def row_body(r, _):
            def vec_body(j, _):
                sl = pl.ds(j * LANES, LANES)
                rows_v[buf, r, sl] = rows_v[buf, r, sl] * SCALE
                return 0

            return lax.fori_loop(0, VECS_PER_ROW, vec_body, 0, unroll=4)

        lax.fori_loop(0, CHUNK, row_body, 0)

    start_gather(0, 0)
    for c in range(N_CHUNKS):
        buf = c % 2
        nxt = (c + 1) % 2
        if c + 1 < N_CHUNKS:
            if c >= 1:
                wait_scatter(c - 1, nxt)
            start_gather(c + 1, nxt)
        wait_gather(c, buf)
        scale_buf(buf)
        pltpu.async_copy(
            rows_v.at[buf],
            out_hbm.at[pl.ds(base + c * CHUNK, CHUNK)],
            out_sems.at[buf],
        )
    wait_scatter(N_CHUNKS - 2, (N_CHUNKS - 2) % 2)
    wait_scatter(N_CHUNKS - 1, (N_CHUNKS - 1) % 2)


@jax.jit
def _embed(table, idx_flat):
    mesh = plsc.VectorSubcoreMesh(core_axis_name="c", subcore_axis_name="s")
    fn = pl.kernel(
        _body,
        out_type=jax.ShapeDtypeStruct((B_TOTAL, D_MODEL), jnp.float32),
        mesh=mesh,
        scratch_types=[
            pltpu.VMEM((B_PER_W,), jnp.int32),
            pltpu.VMEM((2, CHUNK, D_MODEL), jnp.float32),
            pltpu.SemaphoreType.DMA((2,)),
            pltpu.SemaphoreType.DMA((2,)),
        ],
    )
    return fn(table, idx_flat)


def kernel(indices, table):
    idx_flat = indices.reshape(-1).astype(jnp.int32)
    out = _embed(table, idx_flat)
    return out.reshape(indices.shape + (D_MODEL,))


# speedup vs baseline: 1.5439x; 1.0266x over previous
import functools
import math

import jax
import jax.numpy as jnp
from jax import lax
from jax.experimental import pallas as pl
from jax.experimental.pallas import tpu as pltpu
from jax.experimental.pallas import tpu_sc as plsc

D_MODEL = 512
SCALE = math.sqrt(512.0)

NC = 2
NS = 16
LANES = 16
NW = NC * NS

B_TOTAL = 4 * 8192
B_PER_W = B_TOTAL // NW
CHUNK = 64
NBUF = 3
N_CHUNKS = B_PER_W // CHUNK
VECS_PER_ROW = D_MODEL // LANES


def _body(table_hbm, idx_hbm, out_hbm, idx_v, rows_v, in_sems, out_sems):
    wid = lax.axis_index("s") * NC + lax.axis_index("c")
    base = wid * B_PER_W

    pltpu.sync_copy(idx_hbm.at[pl.ds(base, B_PER_W)], idx_v)

    def start_gather(c, buf):
        pltpu.async_copy(
            table_hbm.at[idx_v.at[pl.ds(c * CHUNK, CHUNK)]],
            rows_v.at[buf],
            in_sems.at[buf],
        )

    def wait_gather(c, buf):
        pltpu.make_async_copy(
            table_hbm.at[idx_v.at[pl.ds(c * CHUNK, CHUNK)]],
            rows_v.at[buf],
            in_sems.at[buf],
        ).wait()

    def wait_scatter(c, buf):
        pltpu.make_async_copy(
            rows_v.at[buf],
            out_hbm.at[pl.ds(base + c * CHUNK, CHUNK)],
            out_sems.at[buf],
        ).wait()

    def scale_buf(buf):
        def row_body(r, _):
            for j in range(VECS_PER_ROW):
                sl = slice(j * LANES, (j + 1) * LANES)
                rows_v[buf, r, sl] = rows_v[buf, r, sl] * SCALE
            return 0

        lax.fori_loop(0, CHUNK, row_body, 0)

    start_gather(0, 0)
    start_gather(1, 1)
    for c in range(N_CHUNKS):
        buf = c % NBUF
        if c + 2 < N_CHUNKS:
            nxt = (c + 2) % NBUF
            if c >= 1:
                wait_scatter(c - 1, nxt)
            start_gather(c + 2, nxt)
        wait_gather(c, buf)
        scale_buf(buf)
        pltpu.async_copy(
            rows_v.at[buf],
            out_hbm.at[pl.ds(base + c * CHUNK, CHUNK)],
            out_sems.at[buf],
        )
    for c in range(max(N_CHUNKS - NBUF, 0), N_CHUNKS):
        wait_scatter(c, c % NBUF)


@jax.jit
def _embed(table, idx_flat):
    mesh = plsc.VectorSubcoreMesh(core_axis_name="c", subcore_axis_name="s")
    fn = pl.kernel(
        _body,
        out_type=jax.ShapeDtypeStruct((B_TOTAL, D_MODEL), jnp.float32),
        mesh=mesh,
        scratch_types=[
            pltpu.VMEM((B_PER_W,), jnp.int32),
            pltpu.VMEM((NBUF, CHUNK, D_MODEL), jnp.float32),
            pltpu.SemaphoreType.DMA((NBUF,)),
            pltpu.SemaphoreType.DMA((NBUF,)),
        ],
    )
    return fn(table, idx_flat)


def kernel(indices, table):
    idx_flat = indices.reshape(-1).astype(jnp.int32)
    out = _embed(table, idx_flat)
    return out.reshape(indices.shape + (D_MODEL,))
